# baseline (speedup 1.0000x reference)
import functools

import jax
import jax.numpy as jnp
from jax import lax
from jax.experimental import pallas as pl
from jax.experimental.pallas import tpu as pltpu
from jax.experimental.pallas import tpu_sc as plsc

_N = 10000
_E = 320000
_F_IN = 128
_H = 16
_C = 40

_NC, _NS, _L = 2, 16, 16
_NW = _NC * _NS
_CHUNK_E = 1000
_CPW = 10
_EPW = _CHUNK_E * _CPW
_NP = 10240
_ZR = _NP // _NS

_mesh = plsc.VectorSubcoreMesh(core_axis_name="c", subcore_axis_name="s",
                               num_cores=_NC, num_subcores=_NS)
_sc_params = pltpu.CompilerParams(needs_layout_passes=False,
                                  use_tc_tiling_on_sc=False)

_BCAST_DN = lax.GatherDimensionNumbers(
    offset_dims=(), collapsed_slice_dims=(0,), start_index_map=(0,))


def _lane_bcast(v, t):
    idx = jnp.full((_L, 1), t, jnp.int32)
    return lax.gather(v, idx, _BCAST_DN, slice_sizes=(1,),
                      mode=lax.GatherScatterMode.PROMISE_IN_BOUNDS)


@functools.partial(
    pl.kernel,
    out_type=jax.ShapeDtypeStruct((_NW * _NP,), jnp.float32),
    mesh=_mesh,
    compiler_params=_sc_params,
    scratch_types=[
        pltpu.VMEM((2, _CHUNK_E), jnp.int32),
        pltpu.VMEM((2, _CHUNK_E), jnp.float32),
        pltpu.VMEM((_NP,), jnp.float32),
        pltpu.SemaphoreType.DMA,
    ],
)
def _deg_kernel(dst_hbm, ew_hbm, out_hbm, dst_v, ew_v, acc_v, sem_e):
    c = lax.axis_index("c")
    s = lax.axis_index("s")
    wid = s * _NC + c

    def zbody(i, _):
        acc_v[pl.ds(i * _L, _L)] = jnp.zeros((_L,), jnp.float32)
        return 0

    lax.fori_loop(0, _NP // _L, zbody, 0)

    def start_edges(ch, b):
        base_e = wid * _EPW + ch * _CHUNK_E
        return [
            pltpu.async_copy(dst_hbm.at[pl.ds(base_e, _CHUNK_E)],
                             dst_v.at[b], sem_e),
            pltpu.async_copy(ew_hbm.at[pl.ds(base_e, _CHUNK_E)],
                             ew_v.at[b], sem_e),
        ]

    edges = start_edges(0, 0)
    for ch in range(_CPW):
        b = ch % 2
        for e in edges:
            e.wait()
        if ch + 1 < _CPW:
            edges = start_edges(ch + 1, 1 - b)

        def add_body(g, _, b=b):
            off = g * _L
            d16 = dst_v[b, pl.ds(off, _L)]
            w16 = ew_v[b, pl.ds(off, _L)]
            plsc.addupdate_scatter(acc_v, [d16], w16)
            return 0

        lax.fori_loop(0, _CHUNK_E // _L, add_body, 0)
    pltpu.sync_copy(acc_v, out_hbm.at[pl.ds(wid * _NP, _NP)])


@functools.partial(
    pl.kernel,
    out_type=jax.ShapeDtypeStruct((_NC * _NP, _H), jnp.float32),
    mesh=_mesh,
    compiler_params=_sc_params,
    scratch_types=[
        pltpu.VMEM((_NP,), jnp.float32),
        pltpu.VMEM((2, _CHUNK_E), jnp.int32),
        pltpu.VMEM((3, _CHUNK_E), jnp.int32),
        pltpu.VMEM((2, _CHUNK_E), jnp.float32),
        pltpu.VMEM((3, _CHUNK_E, _H), jnp.float32),
        pltpu.VMEM_SHARED((_NP, _H), jnp.float32),
        pltpu.SemaphoreType.DMA,
        pltpu.SemaphoreType.DMA,
        pltpu.SemaphoreType.DMA,
    ],
)
def _mp_kernel(src_hbm, dst_hbm, ew_hbm, dinv_hbm, hw_hbm, out_hbm,
               dinv_v, src_v, dst_v, ew_v, rows_v, acc_sh,
               sem_e, sem_g, sem_s):
    c = lax.axis_index("c")
    s = lax.axis_index("s")
    wid = s * _NC + c

    def zbody(i, _):
        rows_v[0, i, :] = jnp.zeros((_H,), jnp.float32)
        return 0

    lax.fori_loop(0, _ZR, zbody, 0)
    pltpu.sync_copy(rows_v.at[0, pl.ds(0, _ZR)],
                    acc_sh.at[pl.ds(s * _ZR, _ZR)])
    pltpu.sync_copy(dinv_hbm, dinv_v)
    plsc.subcore_barrier()

    def start_edges(ch):
        base_e = wid * _EPW + ch * _CHUNK_E
        return [
            pltpu.async_copy(src_hbm.at[pl.ds(base_e, _CHUNK_E)],
                             src_v.at[ch % 2], sem_e),
            pltpu.async_copy(dst_hbm.at[pl.ds(base_e, _CHUNK_E)],
                             dst_v.at[ch % 3], sem_e),
            pltpu.async_copy(ew_hbm.at[pl.ds(base_e, _CHUNK_E)],
                             ew_v.at[ch % 2], sem_e),
        ]

    def start_gather(ch):
        return pltpu.async_copy(hw_hbm.at[src_v.at[ch % 2]],
                                rows_v.at[ch % 3], sem_g)

    edges = start_edges(0)
    for e in edges:
        e.wait()
    gather = start_gather(0)
    adds = [None] * _CPW
    for ch in range(_CPW):
        b2 = ch % 2
        b3 = ch % 3
        if ch >= 2:
            adds[ch - 2].wait()
        if ch + 1 < _CPW:
            edges = start_edges(ch + 1)
        gather.wait()
        if ch + 1 < _CPW:
            for e in edges:
                e.wait()
            gather = start_gather(ch + 1)

        def scale_body(g, _, b2=b2, b3=b3):
            off = g * _L
            s16 = src_v[b2, pl.ds(off, _L)]
            d16 = dst_v[b3, pl.ds(off, _L)]
            w16 = ew_v[b2, pl.ds(off, _L)]
            n16 = (plsc.load_gather(dinv_v, [s16]) * w16 *
                   plsc.load_gather(dinv_v, [d16]))
            for t in range(_L):
                r = off + t
                rows_v[b3, r, :] = rows_v[b3, r, :] * _lane_bcast(n16, t)
            return 0

        lax.fori_loop(0, _CHUNK_E // _L, scale_body, 0)

        adds[ch] = pltpu.async_copy(rows_v.at[b3], acc_sh.at[dst_v.at[b3]],
                                    sem_s, add=True)
    adds[_CPW - 2].wait()
    adds[_CPW - 1].wait()
    plsc.subcore_barrier()
    pltpu.sync_copy(acc_sh.at[pl.ds(s * _ZR, _ZR)],
                    rows_v.at[0, pl.ds(0, _ZR)])
    pltpu.sync_copy(rows_v.at[0, pl.ds(0, _ZR)],
                    out_hbm.at[pl.ds(c * _NP + s * _ZR, _ZR)])


def _tc1_body(x_ref, w1_ref, b1_ref, wc1_ref, degp_ref, hw1_ref, dinv_ref):
    h = jnp.maximum(
        jnp.dot(x_ref[...], w1_ref[...], preferred_element_type=jnp.float32)
        + b1_ref[...], 0.0)
    hw1_ref[...] = jnp.dot(h, wc1_ref[...], preferred_element_type=jnp.float32)
    deg = jnp.sum(degp_ref[...], axis=0, keepdims=True) + 1.0
    dinv_ref[...] = lax.rsqrt(deg)


_tc1 = pl.pallas_call(
    _tc1_body,
    out_shape=[
        jax.ShapeDtypeStruct((_N, _H), jnp.float32),
        jax.ShapeDtypeStruct((1, _NP), jnp.float32),
    ],
)


def _tc2_body(aggp_ref, hw_ref, dinvc_ref, b_ref, w_ref, hwn_ref):
    aggp = aggp_ref[...]
    agg = aggp[0, :_N, :] + aggp[1, :_N, :]
    d2 = dinvc_ref[...] * dinvc_ref[...]
    h = jnp.maximum(agg + d2 * hw_ref[...] + b_ref[...], 0.0)
    hwn_ref[...] = jnp.dot(h, w_ref[...], preferred_element_type=jnp.float32)


_tc2 = pl.pallas_call(
    _tc2_body,
    out_shape=jax.ShapeDtypeStruct((_N, _H), jnp.float32),
)


def _tc3_body(aggp_ref, hw_ref, dinvc_ref, b_ref, w2_ref, b2_ref, out_ref):
    aggp = aggp_ref[...]
    agg = aggp[0, :_N, :] + aggp[1, :_N, :]
    d2 = dinvc_ref[...] * dinvc_ref[...]
    h = jnp.maximum(agg + d2 * hw_ref[...] + b_ref[...], 0.0)
    out_ref[...] = (
        jnp.dot(h, w2_ref[...], preferred_element_type=jnp.float32)
        + b2_ref[...])


_tc3 = pl.pallas_call(
    _tc3_body,
    out_shape=jax.ShapeDtypeStruct((_N, _C), jnp.float32),
)


def kernel(x, edge_index, edge_weight, W1, b1, Wc1, bc1, Wc2, bc2, W2, b2):
    src_p = edge_index[0]
    dst_p = edge_index[1]
    ew_p = edge_weight

    degp = _deg_kernel(dst_p, ew_p).reshape(_NW, _NP)
    hw1, dinv2d = _tc1(x, W1, b1.reshape(1, _H), Wc1, degp)
    dinv_flat = dinv2d.reshape(_NP)
    dinv_col = dinv_flat[:_N].reshape(_N, 1)

    agg1 = _mp_kernel(src_p, dst_p, ew_p, dinv_flat, hw1)
    hw2 = _tc2(agg1.reshape(_NC, _NP, _H), hw1, dinv_col,
               bc1.reshape(1, _H), Wc2)
    agg2 = _mp_kernel(src_p, dst_p, ew_p, dinv_flat, hw2)
    out = _tc3(agg2.reshape(_NC, _NP, _H), hw2, dinv_col,
               bc2.reshape(1, _H), W2, b2.reshape(1, _C))
    return out


# speedup vs baseline: 1.0909x; 1.0909x over previous
import functools

import jax
import jax.numpy as jnp
from jax import lax
from jax.experimental import pallas as pl
from jax.experimental.pallas import tpu as pltpu
from jax.experimental.pallas import tpu_sc as plsc

_N = 10000
_E = 320000
_F_IN = 128
_H = 16
_C = 40

_NC, _NS, _L = 2, 16, 16
_NW = _NC * _NS
_CHUNK_E = 2000
_CPW = 5
_EPW = _CHUNK_E * _CPW
_NP = 10240
_ZR = _NP // _NS

_mesh = plsc.VectorSubcoreMesh(core_axis_name="c", subcore_axis_name="s",
                               num_cores=_NC, num_subcores=_NS)
_sc_params = pltpu.CompilerParams(needs_layout_passes=False,
                                  use_tc_tiling_on_sc=False)

_BCAST_DN = lax.GatherDimensionNumbers(
    offset_dims=(), collapsed_slice_dims=(0,), start_index_map=(0,))


def _lane_bcast(v, t):
    idx = jnp.full((_L, 1), t, jnp.int32)
    return lax.gather(v, idx, _BCAST_DN, slice_sizes=(1,),
                      mode=lax.GatherScatterMode.PROMISE_IN_BOUNDS)


@functools.partial(
    pl.kernel,
    out_type=jax.ShapeDtypeStruct((_NW * _NP,), jnp.float32),
    mesh=_mesh,
    compiler_params=_sc_params,
    scratch_types=[
        pltpu.VMEM((2, _CHUNK_E), jnp.int32),
        pltpu.VMEM((2, _CHUNK_E), jnp.float32),
        pltpu.VMEM((_NP,), jnp.float32),
        pltpu.SemaphoreType.DMA,
    ],
)
def _deg_kernel(dst_hbm, ew_hbm, out_hbm, dst_v, ew_v, acc_v, sem_e):
    c = lax.axis_index("c")
    s = lax.axis_index("s")
    wid = s * _NC + c

    def zbody(i, _):
        acc_v[pl.ds(i * _L, _L)] = jnp.zeros((_L,), jnp.float32)
        return 0

    lax.fori_loop(0, _NP // _L, zbody, 0)

    def start_edges(ch, b):
        base_e = wid * _EPW + ch * _CHUNK_E
        return [
            pltpu.async_copy(dst_hbm.at[pl.ds(base_e, _CHUNK_E)],
                             dst_v.at[b], sem_e),
            pltpu.async_copy(ew_hbm.at[pl.ds(base_e, _CHUNK_E)],
                             ew_v.at[b], sem_e),
        ]

    edges = start_edges(0, 0)
    for ch in range(_CPW):
        b = ch % 2
        for e in edges:
            e.wait()
        if ch + 1 < _CPW:
            edges = start_edges(ch + 1, 1 - b)

        def add_body(g, _, b=b):
            off = g * _L
            d16 = dst_v[b, pl.ds(off, _L)]
            w16 = ew_v[b, pl.ds(off, _L)]
            plsc.addupdate_scatter(acc_v, [d16], w16)
            return 0

        lax.fori_loop(0, _CHUNK_E // _L, add_body, 0)
    pltpu.sync_copy(acc_v, out_hbm.at[pl.ds(wid * _NP, _NP)])


@functools.partial(
    pl.kernel,
    out_type=jax.ShapeDtypeStruct((_NC * _NP, _H), jnp.float32),
    mesh=_mesh,
    compiler_params=_sc_params,
    scratch_types=[
        pltpu.VMEM((2, _CHUNK_E), jnp.int32),
        pltpu.VMEM((3, _CHUNK_E), jnp.int32),
        pltpu.VMEM((2, _CHUNK_E), jnp.float32),
        pltpu.VMEM((3, _CHUNK_E, _H), jnp.float32),
        pltpu.VMEM_SHARED((_NP, _H), jnp.float32),
        pltpu.SemaphoreType.DMA,
        pltpu.SemaphoreType.DMA,
        pltpu.SemaphoreType.DMA,
    ],
)
def _mp_kernel(src_hbm, dst_hbm, ew_hbm, hw_hbm, out_hbm,
               src_v, dst_v, ew_v, rows_v, acc_sh,
               sem_e, sem_g, sem_s):
    c = lax.axis_index("c")
    s = lax.axis_index("s")
    wid = s * _NC + c

    def zbody(i, _):
        rows_v[0, i, :] = jnp.zeros((_H,), jnp.float32)
        return 0

    lax.fori_loop(0, _ZR, zbody, 0)
    pltpu.sync_copy(rows_v.at[0, pl.ds(0, _ZR)],
                    acc_sh.at[pl.ds(s * _ZR, _ZR)])
    plsc.subcore_barrier()

    def start_edges(ch):
        base_e = wid * _EPW + ch * _CHUNK_E
        return [
            pltpu.async_copy(src_hbm.at[pl.ds(base_e, _CHUNK_E)],
                             src_v.at[ch % 2], sem_e),
            pltpu.async_copy(dst_hbm.at[pl.ds(base_e, _CHUNK_E)],
                             dst_v.at[ch % 3], sem_e),
            pltpu.async_copy(ew_hbm.at[pl.ds(base_e, _CHUNK_E)],
                             ew_v.at[ch % 2], sem_e),
        ]

    def start_gather(ch):
        return pltpu.async_copy(hw_hbm.at[src_v.at[ch % 2]],
                                rows_v.at[ch % 3], sem_g)

    edges = start_edges(0)
    for e in edges:
        e.wait()
    gather = start_gather(0)
    adds = [None] * _CPW
    for ch in range(_CPW):
        b2 = ch % 2
        b3 = ch % 3
        if ch >= 2:
            adds[ch - 2].wait()
        if ch + 1 < _CPW:
            edges = start_edges(ch + 1)
        gather.wait()
        if ch + 1 < _CPW:
            for e in edges:
                e.wait()
            gather = start_gather(ch + 1)

        def scale_body(g, _, b2=b2, b3=b3):
            off = g * _L
            w16 = ew_v[b2, pl.ds(off, _L)]
            for t in range(_L):
                r = off + t
                rows_v[b3, r, :] = rows_v[b3, r, :] * _lane_bcast(w16, t)
            return 0

        lax.fori_loop(0, _CHUNK_E // _L, scale_body, 0)

        adds[ch] = pltpu.async_copy(rows_v.at[b3], acc_sh.at[dst_v.at[b3]],
                                    sem_s, add=True)
    adds[_CPW - 2].wait()
    adds[_CPW - 1].wait()
    plsc.subcore_barrier()
    pltpu.sync_copy(acc_sh.at[pl.ds(s * _ZR, _ZR)],
                    rows_v.at[0, pl.ds(0, _ZR)])
    pltpu.sync_copy(rows_v.at[0, pl.ds(0, _ZR)],
                    out_hbm.at[pl.ds(c * _NP + s * _ZR, _ZR)])


_ONES32 = jnp.ones((_NW, 1), jnp.float32)


def _tc1_body(x_ref, w1_ref, b1_ref, wc1_ref, degp_ref, ones_ref,
              hw1_ref, dinv_ref):
    h = jnp.maximum(
        jnp.dot(x_ref[...], w1_ref[...], preferred_element_type=jnp.float32)
        + b1_ref[...], 0.0)
    deg = jnp.einsum("wn,wo->no", degp_ref[...], ones_ref[...],
                     preferred_element_type=jnp.float32) + 1.0
    dinv = lax.rsqrt(deg)
    dinv_ref[...] = dinv
    hw1_ref[...] = dinv[:_N] * jnp.dot(h, wc1_ref[...],
                                       preferred_element_type=jnp.float32)


_tc1 = pl.pallas_call(
    _tc1_body,
    out_shape=[
        jax.ShapeDtypeStruct((_N, _H), jnp.float32),
        jax.ShapeDtypeStruct((_NP, 1), jnp.float32),
    ],
)


def _tc2_body(aggp_ref, hw_ref, dinvc_ref, b_ref, w_ref, hwn_ref):
    aggp = aggp_ref[...]
    agg = aggp[0, :_N, :] + aggp[1, :_N, :]
    dinvc = dinvc_ref[...]
    h = jnp.maximum(dinvc * (agg + hw_ref[...]) + b_ref[...], 0.0)
    hwn_ref[...] = dinvc * jnp.dot(h, w_ref[...],
                                   preferred_element_type=jnp.float32)


_tc2 = pl.pallas_call(
    _tc2_body,
    out_shape=jax.ShapeDtypeStruct((_N, _H), jnp.float32),
)


def _tc3_body(aggp_ref, hw_ref, dinvc_ref, b_ref, w2_ref, b2_ref, out_ref):
    aggp = aggp_ref[...]
    agg = aggp[0, :_N, :] + aggp[1, :_N, :]
    dinvc = dinvc_ref[...]
    h = jnp.maximum(dinvc * (agg + hw_ref[...]) + b_ref[...], 0.0)
    out_ref[...] = (
        jnp.dot(h, w2_ref[...], preferred_element_type=jnp.float32)
        + b2_ref[...])


_tc3 = pl.pallas_call(
    _tc3_body,
    out_shape=jax.ShapeDtypeStruct((_N, _C), jnp.float32),
)


def kernel(x, edge_index, edge_weight, W1, b1, Wc1, bc1, Wc2, bc2, W2, b2):
    src_p = edge_index[0]
    dst_p = edge_index[1]
    ew_p = edge_weight

    degp = _deg_kernel(dst_p, ew_p).reshape(_NW, _NP)
    hw1, dinv_col_p = _tc1(x, W1, b1.reshape(1, _H), Wc1, degp, _ONES32)
    dinv_col = dinv_col_p[:_N]

    agg1 = _mp_kernel(src_p, dst_p, ew_p, hw1)
    hw2 = _tc2(agg1.reshape(_NC, _NP, _H), hw1, dinv_col,
               bc1.reshape(1, _H), Wc2)
    agg2 = _mp_kernel(src_p, dst_p, ew_p, hw2)
    out = _tc3(agg2.reshape(_NC, _NP, _H), hw2, dinv_col,
               bc2.reshape(1, _H), W2, b2.reshape(1, _C))
    return out
